# baseline (speedup 1.0000x reference)
import jax
import jax.numpy as jnp
from jax import lax
from jax.experimental import pallas as pl
from jax.experimental.pallas import tpu as pltpu
from jax.experimental.pallas import tpu_sc as plsc

_NC = 2
_NS = 16
_NW = _NC * _NS
_L = 16
_R = 4
_NBUF = 8
_K = 4

_TC_BATCHES = 2
_TC_RB = 512


def _sc_body(x_hbm, pos_hbm, out_hbm, *scratch):
    xbufs = scratch[0:_NBUF]
    pbufs = scratch[_NBUF:2 * _NBUF]
    lsems = scratch[2 * _NBUF:3 * _NBUF]
    ssems = scratch[3 * _NBUF:4 * _NBUF]

    nrows_x, d = x_hbm.shape
    nrows_sc = out_hbm.shape[0]
    row0 = nrows_x - nrows_sc
    wid = lax.axis_index("s") * _NC + lax.axis_index("c")
    rows_per_w = nrows_sc // _NW
    base_out = wid * rows_per_w
    base_x = row0 + base_out
    pbase = lax.rem(base_x, pos_hbm.shape[0])
    nchunks = rows_per_w // _R

    def issue_load(c, t):
        off = c * _R
        pltpu.async_copy(x_hbm.at[pl.ds(base_x + off, _R)], xbufs[t], lsems[t])
        pltpu.async_copy(pos_hbm.at[pl.ds(pbase + off, _R)], pbufs[t], lsems[t])

    def wait_load(s):
        pltpu.make_async_copy(x_hbm.at[pl.ds(0, _R)], xbufs[s], lsems[s]).wait()
        pltpu.make_async_copy(pos_hbm.at[pl.ds(0, _R)], pbufs[s], lsems[s]).wait()

    def wait_store(t):
        pltpu.make_async_copy(xbufs[t], out_hbm.at[pl.ds(0, _R)], ssems[t]).wait()

    def chunk_body(c, s, do_wait_store, do_load):
        t = (s + _K) % _NBUF
        if do_wait_store:
            wait_store(t)
        if do_load:
            issue_load(c + _K, t)
        wait_load(s)

        xb, pb = xbufs[s], pbufs[s]
        for r in range(_R):
            @plsc.parallel_loop(0, d, step=_L, unroll=8)
            def _col(i):
                sl = pl.ds(i, _L)
                plsc.addupdate(xb.at[r, sl], pb[r, sl])

        pltpu.async_copy(xb, out_hbm.at[pl.ds(base_out + c * _R, _R)],
                         ssems[s])

    for c in range(_K):
        issue_load(c, c % _NBUF)

    for s in range(_NBUF):
        chunk_body(s, s, do_wait_store=(s >= _NBUF - _K), do_load=True)

    def outer(g, carry):
        for s in range(_NBUF):
            chunk_body(g * _NBUF + s, s, do_wait_store=True, do_load=True)
        return carry

    lax.fori_loop(1, nchunks // _NBUF - 1, outer, 0)

    for s in range(_NBUF):
        c = nchunks - _NBUF + s
        chunk_body(c, s, do_wait_store=True, do_load=(s < _NBUF - _K))

    for s in range(_NBUF - _K, _NBUF):
        wait_store(s)


def _tc_body(x_ref, pos_ref, o_ref):
    o_ref[...] = x_ref[...] + pos_ref[...]


def kernel(x, pos_table):
    B, S, D = x.shape
    xr = x.reshape(B * S, D)
    pos = pos_table[:S]

    n_tc = _TC_BATCHES * S
    tc_out = pl.pallas_call(
        _tc_body,
        grid=(n_tc // _TC_RB,),
        in_specs=[
            pl.BlockSpec((_TC_RB, D), lambda i: (i, 0)),
            pl.BlockSpec((_TC_RB, D), lambda i: (i % (S // _TC_RB), 0)),
        ],
        out_specs=pl.BlockSpec((_TC_RB, D), lambda i: (i, 0)),
        out_shape=jax.ShapeDtypeStruct((n_tc, D), jnp.float32),
    )(xr, pos)

    mesh = plsc.VectorSubcoreMesh(core_axis_name="c", subcore_axis_name="s")
    sc_out = pl.kernel(
        _sc_body,
        out_type=jax.ShapeDtypeStruct((B * S - n_tc, D), jnp.float32),
        mesh=mesh,
        scratch_types=(
            [pltpu.VMEM((_R, D), jnp.float32)] * (2 * _NBUF)
            + [pltpu.SemaphoreType.DMA] * (2 * _NBUF)
        ),
    )(xr, pos)

    out = jnp.concatenate([tc_out, sc_out], axis=0)
    return out.reshape(B, S, D)


# speedup vs baseline: 1.0167x; 1.0167x over previous
import jax
import jax.numpy as jnp
from jax import lax
from jax.experimental import pallas as pl
from jax.experimental.pallas import tpu as pltpu
from jax.experimental.pallas import tpu_sc as plsc

_NC = 2
_NS = 16
_NW = _NC * _NS
_L = 16
_R = 4
_NBUF = 8
_K = 4

_TC_BATCHES = 3
_TC_RB = 512


def _sc_body(x_hbm, pos_hbm, out_hbm, *scratch):
    xbufs = scratch[0:_NBUF]
    pbufs = scratch[_NBUF:2 * _NBUF]
    lsems = scratch[2 * _NBUF:3 * _NBUF]
    ssems = scratch[3 * _NBUF:4 * _NBUF]

    nrows_x, d = x_hbm.shape
    nrows_sc = out_hbm.shape[0]
    row0 = nrows_x - nrows_sc
    wid = lax.axis_index("s") * _NC + lax.axis_index("c")
    rows_per_w = nrows_sc // _NW
    base_out = wid * rows_per_w
    base_x = row0 + base_out
    pbase = lax.rem(base_x, pos_hbm.shape[0])
    nchunks = rows_per_w // _R

    def issue_load(c, t):
        off = c * _R
        pltpu.async_copy(x_hbm.at[pl.ds(base_x + off, _R)], xbufs[t], lsems[t])
        pltpu.async_copy(pos_hbm.at[pl.ds(pbase + off, _R)], pbufs[t], lsems[t])

    def wait_load(s):
        pltpu.make_async_copy(x_hbm.at[pl.ds(0, _R)], xbufs[s], lsems[s]).wait()
        pltpu.make_async_copy(pos_hbm.at[pl.ds(0, _R)], pbufs[s], lsems[s]).wait()

    def wait_store(t):
        pltpu.make_async_copy(xbufs[t], out_hbm.at[pl.ds(0, _R)], ssems[t]).wait()

    def chunk_body(c, s, do_wait_store, do_load):
        t = (s + _K) % _NBUF
        if do_wait_store:
            wait_store(t)
        if do_load:
            issue_load(c + _K, t)
        wait_load(s)

        xb, pb = xbufs[s], pbufs[s]
        for r in range(_R):
            @plsc.parallel_loop(0, d, step=_L, unroll=8)
            def _col(i):
                sl = pl.ds(i, _L)
                plsc.addupdate(xb.at[r, sl], pb[r, sl])

        pltpu.async_copy(xb, out_hbm.at[pl.ds(base_out + c * _R, _R)],
                         ssems[s])

    for c in range(_K):
        issue_load(c, c % _NBUF)

    for s in range(_NBUF):
        chunk_body(s, s, do_wait_store=(s >= _NBUF - _K), do_load=True)

    def outer(g, carry):
        for s in range(_NBUF):
            chunk_body(g * _NBUF + s, s, do_wait_store=True, do_load=True)
        return carry

    lax.fori_loop(1, nchunks // _NBUF - 1, outer, 0)

    for s in range(_NBUF):
        c = nchunks - _NBUF + s
        chunk_body(c, s, do_wait_store=True, do_load=(s < _NBUF - _K))

    for s in range(_NBUF - _K, _NBUF):
        wait_store(s)


def _tc_body(x_ref, pos_ref, o_ref):
    o_ref[...] = x_ref[...] + pos_ref[...]


def kernel(x, pos_table):
    B, S, D = x.shape
    xr = x.reshape(B * S, D)
    pos = pos_table[:S]

    n_tc = _TC_BATCHES * S
    tc_out = pl.pallas_call(
        _tc_body,
        grid=(n_tc // _TC_RB,),
        in_specs=[
            pl.BlockSpec((_TC_RB, D), lambda i: (i, 0)),
            pl.BlockSpec((_TC_RB, D), lambda i: (i % (S // _TC_RB), 0)),
        ],
        out_specs=pl.BlockSpec((_TC_RB, D), lambda i: (i, 0)),
        out_shape=jax.ShapeDtypeStruct((n_tc, D), jnp.float32),
    )(xr, pos)

    mesh = plsc.VectorSubcoreMesh(core_axis_name="c", subcore_axis_name="s")
    sc_out = pl.kernel(
        _sc_body,
        out_type=jax.ShapeDtypeStruct((B * S - n_tc, D), jnp.float32),
        mesh=mesh,
        scratch_types=(
            [pltpu.VMEM((_R, D), jnp.float32)] * (2 * _NBUF)
            + [pltpu.SemaphoreType.DMA] * (2 * _NBUF)
        ),
    )(xr, pos)

    out = jnp.concatenate([tc_out, sc_out], axis=0)
    return out.reshape(B, S, D)


# speedup vs baseline: 1.4050x; 1.3819x over previous
import jax
import jax.numpy as jnp
from jax import lax
from jax.experimental import pallas as pl
from jax.experimental.pallas import tpu as pltpu
from jax.experimental.pallas import tpu_sc as plsc

_NC = 2
_NS = 16
_NW = _NC * _NS
_L = 16
_R = 4
_NBUF = 8
_K = 4


def _sc_body(x_hbm, pos_hbm, out_hbm, *scratch):
    xbufs = scratch[0:_NBUF]
    pbufs = scratch[_NBUF:2 * _NBUF]
    lsems = scratch[2 * _NBUF:3 * _NBUF]
    ssems = scratch[3 * _NBUF:4 * _NBUF]

    nrows, d = x_hbm.shape
    wid = lax.axis_index("s") * _NC + lax.axis_index("c")
    rows_per_w = nrows // _NW
    workers_per_batch = pos_hbm.shape[0] // rows_per_w
    base = wid * rows_per_w
    pbase = lax.rem(wid, workers_per_batch) * rows_per_w
    nchunks = rows_per_w // _R

    def issue_load(c, t):
        off = c * _R
        pltpu.async_copy(x_hbm.at[pl.ds(base + off, _R)], xbufs[t], lsems[t])
        pltpu.async_copy(pos_hbm.at[pl.ds(pbase + off, _R)], pbufs[t], lsems[t])

    def wait_load(s):
        pltpu.make_async_copy(x_hbm.at[pl.ds(0, _R)], xbufs[s], lsems[s]).wait()
        pltpu.make_async_copy(pos_hbm.at[pl.ds(0, _R)], pbufs[s], lsems[s]).wait()

    def wait_store(t):
        pltpu.make_async_copy(xbufs[t], out_hbm.at[pl.ds(0, _R)], ssems[t]).wait()

    def chunk_body(c, s, do_wait_store, do_load):
        t = (s + _K) % _NBUF
        if do_wait_store:
            wait_store(t)
        if do_load:
            issue_load(c + _K, t)
        wait_load(s)

        xb, pb = xbufs[s], pbufs[s]
        for r in range(_R):
            @plsc.parallel_loop(0, d, step=_L, unroll=8)
            def _col(i):
                sl = pl.ds(i, _L)
                plsc.addupdate(xb.at[r, sl], pb[r, sl])

        pltpu.async_copy(xb, out_hbm.at[pl.ds(base + c * _R, _R)], ssems[s])

    for c in range(_K):
        issue_load(c, c % _NBUF)

    for s in range(_NBUF):
        chunk_body(s, s, do_wait_store=(s >= _NBUF - _K), do_load=True)

    def outer(g, carry):
        for s in range(_NBUF):
            chunk_body(g * _NBUF + s, s, do_wait_store=True, do_load=True)
        return carry

    lax.fori_loop(1, nchunks // _NBUF - 1, outer, 0)

    for s in range(_NBUF):
        c = nchunks - _NBUF + s
        chunk_body(c, s, do_wait_store=True, do_load=(s < _NBUF - _K))

    for s in range(_NBUF - _K, _NBUF):
        wait_store(s)


def kernel(x, pos_table):
    B, S, D = x.shape
    xr = x.reshape(B * S, D)
    mesh = plsc.VectorSubcoreMesh(core_axis_name="c", subcore_axis_name="s")
    out = pl.kernel(
        _sc_body,
        out_type=jax.ShapeDtypeStruct((B * S, D), jnp.float32),
        mesh=mesh,
        scratch_types=(
            [pltpu.VMEM((_R, D), jnp.float32)] * (2 * _NBUF)
            + [pltpu.SemaphoreType.DMA] * (2 * _NBUF)
        ),
    )(xr, pos_table[:S])
    return out.reshape(B, S, D)
